# Initial kernel scaffold; baseline (speedup 1.0000x reference)
import jax
import jax.numpy as jnp
from jax.experimental import pallas as pl


def kernel(x, edge_index, edge_feat, rel_table, Wq, bq, Wk, Wv, Wo, bo, W1, b1, W2, b2, ln1_g, ln1_b, ln2_g, ln2_b):
    raise NotImplementedError("write your pallas kernel here")


# speedup vs baseline: 12.8360x; 12.8360x over previous
import functools
import math

import jax
import jax.numpy as jnp
from jax import lax
from jax.experimental import pallas as pl
from jax.experimental.pallas import tpu as pltpu
from jax.experimental.pallas import tpu_sc as plsc

N = 10000
E = 320000
HID = 128
HEADS = 8
DK = 16
REL = 256
FF = 512

NWORK = 32
EPW = E // NWORK
CHUNK = 80
NCHUNK = EPW // CHUNK
GROUPS = CHUNK // 16
ROWS_PER_SUB = 624
ROWS_TAIL = N - 16 * ROWS_PER_SUB

_BLK = 1000
_GRID = N // _BLK


def _qkv_body(x_ref, wq_ref, bq_ref, wkv_ref, q_ref, kv_ref):
    xb = x_ref[...]
    q_ref[...] = (
        jnp.dot(xb, wq_ref[...], preferred_element_type=jnp.float32)
        + bq_ref[...]
    )
    kv_ref[...] = jnp.dot(xb, wkv_ref[...], preferred_element_type=jnp.float32)


def _qkv(x, Wq, bq, Wkv):
    return pl.pallas_call(
        _qkv_body,
        grid=(_GRID,),
        in_specs=[
            pl.BlockSpec((_BLK, HID), lambda i: (i, 0)),
            pl.BlockSpec((HID, HID), lambda i: (0, 0)),
            pl.BlockSpec((1, HID), lambda i: (0, 0)),
            pl.BlockSpec((HID, 2 * HID), lambda i: (0, 0)),
        ],
        out_specs=[
            pl.BlockSpec((_BLK, HID), lambda i: (i, 0)),
            pl.BlockSpec((_BLK, 2 * HID), lambda i: (i, 0)),
        ],
        out_shape=[
            jax.ShapeDtypeStruct((N, HID), jnp.float32),
            jax.ShapeDtypeStruct((N, 2 * HID), jnp.float32),
        ],
    )(x, Wq, bq, Wkv)


def _edge_body(
    kv_hbm, q_hbm, rel_hbm, src_hbm, dst_hbm, feat_hbm, zwv_hbm, zz_hbm,
    lg_out, wv_out, z_out,
    src_b, dst_b, feat_b, kv_b, q_b, lg_b, msg_b, sco_b,
    acc_wv, acc_z, sem0, sem1, sem2,
):
    c = lax.axis_index("c")
    s = lax.axis_index("s")
    wid = s * 2 + c

    pltpu.sync_copy(zwv_hbm, acc_wv.at[pl.ds(s * ROWS_PER_SUB, ROWS_PER_SUB)])
    pltpu.sync_copy(zz_hbm, acc_z.at[pl.ds(s * ROWS_PER_SUB, ROWS_PER_SUB)])

    @pl.when(s == 15)
    def _zero_tail():
        pltpu.sync_copy(
            zwv_hbm.at[pl.ds(0, ROWS_TAIL)],
            acc_wv.at[pl.ds(16 * ROWS_PER_SUB, ROWS_TAIL)],
        )
        pltpu.sync_copy(
            zz_hbm.at[pl.ds(0, ROWS_TAIL)],
            acc_z.at[pl.ds(16 * ROWS_PER_SUB, ROWS_TAIL)],
        )
    plsc.subcore_barrier()

    iota = lax.broadcasted_iota(jnp.int32, (16,), 0)

    def chunk_step(j, carry):
        pltpu.sync_copy(src_hbm.at[wid, j], src_b)
        pltpu.sync_copy(dst_hbm.at[wid, j], dst_b)
        pltpu.sync_copy(feat_hbm.at[wid, j], feat_b)
        cp0 = pltpu.async_copy(kv_hbm.at[src_b], kv_b, sem0)
        cp1 = pltpu.async_copy(q_hbm.at[dst_b], q_b, sem1)
        cp2 = pltpu.async_copy(rel_hbm.at[feat_b], lg_b, sem2)
        cp0.wait()
        cp1.wait()
        cp2.wait()
        pltpu.sync_copy(lg_b, lg_out.at[wid * NCHUNK + j])

        def group_step(g, gcarry):
            rows = g * 16 + iota
            accs = [jnp.zeros((16,), jnp.float32) for _ in range(HEADS)]
            lgs = []
            for d in range(DK):
                lgv = plsc.load_gather(lg_b, [rows, jnp.full((16,), d, jnp.int32)])
                lgs.append(lgv)
                for h in range(HEADS):
                    col = jnp.full((16,), h * DK + d, jnp.int32)
                    kvv = plsc.load_gather(kv_b, [rows, col])
                    qv = plsc.load_gather(q_b, [rows, col])
                    accs[h] = accs[h] + (kvv + lgv) * qv
            ss = []
            for h in range(HEADS):
                sc = jnp.exp(jnp.clip(accs[h] * 0.25, -10.0, 10.0))
                ss.append(sc)
                plsc.store_scatter(
                    sco_b, [rows, jnp.full((16,), h, jnp.int32)], sc
                )
            for d in range(DK):
                for h in range(HEADS):
                    vcol = jnp.full((16,), HID + h * DK + d, jnp.int32)
                    vv = plsc.load_gather(kv_b, [rows, vcol])
                    m = (vv + lgs[d]) * ss[h]
                    plsc.store_scatter(
                        msg_b, [rows, jnp.full((16,), h * DK + d, jnp.int32)], m
                    )
            return gcarry

        lax.fori_loop(0, GROUPS, group_step, 0)
        pltpu.sync_copy(msg_b, acc_wv.at[dst_b], add=True)
        pltpu.sync_copy(sco_b, acc_z.at[dst_b], add=True)
        return carry

    lax.fori_loop(0, NCHUNK, chunk_step, 0)
    plsc.subcore_barrier()
    rs = pl.ds(s * ROWS_PER_SUB, ROWS_PER_SUB)
    pltpu.sync_copy(acc_wv.at[rs], wv_out.at[c, rs])
    pltpu.sync_copy(acc_z.at[rs], z_out.at[c, rs])

    @pl.when(s == 15)
    def _dump_tail():
        rt = pl.ds(16 * ROWS_PER_SUB, ROWS_TAIL)
        pltpu.sync_copy(acc_wv.at[rt], wv_out.at[c, rt])
        pltpu.sync_copy(acc_z.at[rt], z_out.at[c, rt])


def _edge(kv_tab, q_tab, rel_table, src3, dst3, feat3, zwv, zz):
    mesh = plsc.VectorSubcoreMesh(core_axis_name="c", subcore_axis_name="s")
    f = pl.kernel(
        _edge_body,
        out_type=[
            jax.ShapeDtypeStruct((NWORK * NCHUNK, CHUNK, DK), jnp.float32),
            jax.ShapeDtypeStruct((2, N, HID), jnp.float32),
            jax.ShapeDtypeStruct((2, N, HEADS), jnp.float32),
        ],
        mesh=mesh,
        compiler_params=pltpu.CompilerParams(
            needs_layout_passes=False, use_tc_tiling_on_sc=False
        ),
        scratch_types=[
            pltpu.VMEM((CHUNK,), jnp.int32),
            pltpu.VMEM((CHUNK,), jnp.int32),
            pltpu.VMEM((CHUNK,), jnp.int32),
            pltpu.VMEM((CHUNK, 2 * HID), jnp.float32),
            pltpu.VMEM((CHUNK, HID), jnp.float32),
            pltpu.VMEM((CHUNK, DK), jnp.float32),
            pltpu.VMEM((CHUNK, HID), jnp.float32),
            pltpu.VMEM((CHUNK, HEADS), jnp.float32),
            pltpu.VMEM_SHARED((N, HID), jnp.float32),
            pltpu.VMEM_SHARED((N, HEADS), jnp.float32),
            pltpu.SemaphoreType.DMA,
            pltpu.SemaphoreType.DMA,
            pltpu.SemaphoreType.DMA,
        ],
    )
    return f(kv_tab, q_tab, rel_table, src3, dst3, feat3, zwv, zz)


def _post_body(
    wv_ref, z_ref, x_ref, r_ref, wo_ref, bo_ref, w1_ref, b1_ref, w2_ref,
    b2_ref, g1_ref, be1_ref, g2_ref, be2_ref, out_ref,
):
    acc = wv_ref[0] + wv_ref[1]
    zz = z_ref[0] + z_ref[1]
    zrep = jnp.dot(1.0 / zz, r_ref[...], preferred_element_type=jnp.float32)
    o = acc * zrep
    h1 = (
        x_ref[...]
        + jnp.dot(o, wo_ref[...], preferred_element_type=jnp.float32)
        + bo_ref[...]
    )
    m1 = jnp.mean(h1, axis=1, keepdims=True)
    v1 = jnp.mean((h1 - m1) ** 2, axis=1, keepdims=True)
    out1 = g1_ref[...] * (h1 - m1) / jnp.sqrt(v1 + 1e-5) + be1_ref[...]
    ff = jnp.maximum(
        jnp.dot(out1, w1_ref[...], preferred_element_type=jnp.float32)
        + b1_ref[...],
        0.0,
    )
    h2 = (
        out1
        + jnp.dot(ff, w2_ref[...], preferred_element_type=jnp.float32)
        + b2_ref[...]
    )
    m2 = jnp.mean(h2, axis=1, keepdims=True)
    v2 = jnp.mean((h2 - m2) ** 2, axis=1, keepdims=True)
    out_ref[...] = g2_ref[...] * (h2 - m2) / jnp.sqrt(v2 + 1e-5) + be2_ref[...]


def _post(wv2, z2, x, R, Wo, bo, W1, b1, W2, b2, g1, be1, g2, be2):
    full = lambda shape: pl.BlockSpec(shape, lambda i, _s=shape: tuple(0 for _ in _s))
    return pl.pallas_call(
        _post_body,
        grid=(_GRID,),
        in_specs=[
            pl.BlockSpec((2, _BLK, HID), lambda i: (0, i, 0)),
            pl.BlockSpec((2, _BLK, HEADS), lambda i: (0, i, 0)),
            pl.BlockSpec((_BLK, HID), lambda i: (i, 0)),
            full((HEADS, HID)),
            full((HID, HID)),
            full((1, HID)),
            full((HID, FF)),
            full((1, FF)),
            full((FF, HID)),
            full((1, HID)),
            full((1, HID)),
            full((1, HID)),
            full((1, HID)),
            full((1, HID)),
        ],
        out_specs=pl.BlockSpec((_BLK, HID), lambda i: (i, 0)),
        out_shape=jax.ShapeDtypeStruct((N, HID), jnp.float32),
    )(wv2, z2, x, R, Wo, bo, W1, b1, W2, b2, g1, be1, g2, be2)


def kernel(x, edge_index, edge_feat, rel_table, Wq, bq, Wk, Wv, Wo, bo,
           W1, b1, W2, b2, ln1_g, ln1_b, ln2_g, ln2_b):
    ei = edge_index.astype(jnp.int32)
    ef = edge_feat.astype(jnp.int32)
    src3 = ei[0].reshape(NWORK, NCHUNK, CHUNK)
    dst3 = ei[1].reshape(NWORK, NCHUNK, CHUNK)
    feat3 = ef.reshape(NWORK, NCHUNK, CHUNK)
    Wkv = jnp.concatenate([Wk, Wv], axis=1)

    q_tab, kv_tab = _qkv(x, Wq, bq.reshape(1, HID), Wkv)

    zwv = jnp.zeros((ROWS_PER_SUB, HID), jnp.float32)
    zz = jnp.zeros((ROWS_PER_SUB, HEADS), jnp.float32)
    lg4, wv2, z2 = _edge(kv_tab, q_tab, rel_table, src3, dst3, feat3, zwv, zz)
    lg_x = lg4.reshape(E, DK)

    R = jnp.repeat(jnp.eye(HEADS, dtype=jnp.float32), DK, axis=1)
    out_x = _post(
        wv2, z2, x, R, Wo, bo.reshape(1, HID), W1, b1.reshape(1, FF),
        W2, b2.reshape(1, HID), ln1_g.reshape(1, HID), ln1_b.reshape(1, HID),
        ln2_g.reshape(1, HID), ln2_b.reshape(1, HID),
    )
    return (out_x, lg_x)


# speedup vs baseline: 18.3169x; 1.4270x over previous
import functools
import math

import jax
import jax.numpy as jnp
from jax import lax
from jax.experimental import pallas as pl
from jax.experimental.pallas import tpu as pltpu
from jax.experimental.pallas import tpu_sc as plsc

N = 10000
E = 320000
HID = 128
HEADS = 8
DK = 16
REL = 256
FF = 512

NWORK = 32
EPW = E // NWORK
CHUNK = 80
NCHUNK = EPW // CHUNK
GROUPS = CHUNK // 16
ROWS_PER_SUB = 624
ROWS_TAIL = N - 16 * ROWS_PER_SUB

_BLK = 1000
_GRID = N // _BLK


def _qkv_body(x_ref, wq_ref, bq_ref, wkv_ref, q_ref, kv_ref):
    xb = x_ref[...]
    q_ref[...] = (
        jnp.dot(xb, wq_ref[...], preferred_element_type=jnp.float32)
        + bq_ref[...]
    )
    kv_ref[...] = jnp.dot(xb, wkv_ref[...], preferred_element_type=jnp.float32)


def _qkv(x, Wq, bq, Wkv):
    return pl.pallas_call(
        _qkv_body,
        grid=(_GRID,),
        in_specs=[
            pl.BlockSpec((_BLK, HID), lambda i: (i, 0)),
            pl.BlockSpec((HID, HID), lambda i: (0, 0)),
            pl.BlockSpec((1, HID), lambda i: (0, 0)),
            pl.BlockSpec((HID, 2 * HID), lambda i: (0, 0)),
        ],
        out_specs=[
            pl.BlockSpec((_BLK, HID), lambda i: (i, 0)),
            pl.BlockSpec((_BLK, 2 * HID), lambda i: (i, 0)),
        ],
        out_shape=[
            jax.ShapeDtypeStruct((N, HID), jnp.float32),
            jax.ShapeDtypeStruct((N, 2 * HID), jnp.float32),
        ],
    )(x, Wq, bq, Wkv)


def _edge_body(
    kv_hbm, q_hbm, rel_hbm, src_hbm, dst_hbm, feat_hbm, zwv_hbm, zz_hbm,
    lg_out, wv_out, z_out,
    src_b, dst_b, feat_b, kv_b, q_b, lg_b, msg_b, sco_b, tp_b,
    acc_wv, acc_z, sem0, sem1, sem2,
):
    c = lax.axis_index("c")
    s = lax.axis_index("s")
    wid = s * 2 + c

    pltpu.sync_copy(zwv_hbm, acc_wv.at[pl.ds(s * ROWS_PER_SUB, ROWS_PER_SUB)])
    pltpu.sync_copy(zz_hbm, acc_z.at[pl.ds(s * ROWS_PER_SUB, ROWS_PER_SUB)])

    @pl.when(s == 15)
    def _zero_tail():
        pltpu.sync_copy(
            zwv_hbm.at[pl.ds(0, ROWS_TAIL)],
            acc_wv.at[pl.ds(16 * ROWS_PER_SUB, ROWS_TAIL)],
        )
        pltpu.sync_copy(
            zz_hbm.at[pl.ds(0, ROWS_TAIL)],
            acc_z.at[pl.ds(16 * ROWS_PER_SUB, ROWS_TAIL)],
        )
    plsc.subcore_barrier()

    iota = lax.broadcasted_iota(jnp.int32, (16,), 0)

    def chunk_step(j, carry):
        pltpu.sync_copy(src_hbm.at[wid, j], src_b)
        pltpu.sync_copy(dst_hbm.at[wid, j], dst_b)
        pltpu.sync_copy(feat_hbm.at[wid, j], feat_b)
        cp0 = pltpu.async_copy(kv_hbm.at[src_b], kv_b, sem0)
        cp1 = pltpu.async_copy(q_hbm.at[dst_b], q_b, sem1)
        cp2 = pltpu.async_copy(rel_hbm.at[feat_b], lg_b, sem2)
        cp0.wait()
        cp1.wait()
        cp2.wait()
        pltpu.sync_copy(lg_b, lg_out.at[wid * NCHUNK + j])

        def group_step(g, gcarry):
            base = g * 16
            rows = base + iota
            lgs = [lg_b[base + e, :] for e in range(16)]
            ss = []
            for h in range(HEADS):
                for e in range(16):
                    ke = kv_b[base + e, pl.ds(h * DK, DK)]
                    qe = q_b[base + e, pl.ds(h * DK, DK)]
                    tp_b[e, pl.ds(0, DK)] = (ke + lgs[e]) * qe
                acc = jnp.zeros((16,), jnp.float32)
                for d in range(DK):
                    acc = acc + plsc.load_gather(
                        tp_b, [iota, jnp.full((16,), d, jnp.int32)]
                    )
                s_h = jnp.exp(jnp.clip(acc * 0.25, -10.0, 10.0))
                ss.append(s_h)
                plsc.store_scatter(
                    sco_b, [rows, jnp.full((16,), h, jnp.int32)], s_h
                )
            for e in range(16):
                for h in range(HEADS):
                    ve = kv_b[base + e, pl.ds(HID + h * DK, DK)]
                    msg_b[base + e, pl.ds(h * DK, DK)] = (ve + lgs[e]) * ss[h][e]
            return gcarry

        lax.fori_loop(0, GROUPS, group_step, 0)
        pltpu.sync_copy(msg_b, acc_wv.at[dst_b], add=True)
        pltpu.sync_copy(sco_b, acc_z.at[dst_b], add=True)
        return carry

    lax.fori_loop(0, NCHUNK, chunk_step, 0)
    plsc.subcore_barrier()
    rs = pl.ds(s * ROWS_PER_SUB, ROWS_PER_SUB)
    pltpu.sync_copy(acc_wv.at[rs], wv_out.at[c, rs])
    pltpu.sync_copy(acc_z.at[rs], z_out.at[c, rs])

    @pl.when(s == 15)
    def _dump_tail():
        rt = pl.ds(16 * ROWS_PER_SUB, ROWS_TAIL)
        pltpu.sync_copy(acc_wv.at[rt], wv_out.at[c, rt])
        pltpu.sync_copy(acc_z.at[rt], z_out.at[c, rt])


def _edge(kv_tab, q_tab, rel_table, src3, dst3, feat3, zwv, zz):
    mesh = plsc.VectorSubcoreMesh(core_axis_name="c", subcore_axis_name="s")
    f = pl.kernel(
        _edge_body,
        out_type=[
            jax.ShapeDtypeStruct((NWORK * NCHUNK, CHUNK, DK), jnp.float32),
            jax.ShapeDtypeStruct((2, N, HID), jnp.float32),
            jax.ShapeDtypeStruct((2, N, HEADS), jnp.float32),
        ],
        mesh=mesh,
        compiler_params=pltpu.CompilerParams(
            needs_layout_passes=False, use_tc_tiling_on_sc=False
        ),
        scratch_types=[
            pltpu.VMEM((CHUNK,), jnp.int32),
            pltpu.VMEM((CHUNK,), jnp.int32),
            pltpu.VMEM((CHUNK,), jnp.int32),
            pltpu.VMEM((CHUNK, 2 * HID), jnp.float32),
            pltpu.VMEM((CHUNK, HID), jnp.float32),
            pltpu.VMEM((CHUNK, DK), jnp.float32),
            pltpu.VMEM((CHUNK, HID), jnp.float32),
            pltpu.VMEM((CHUNK, HEADS), jnp.float32),
            pltpu.VMEM((16, DK + 1), jnp.float32),
            pltpu.VMEM_SHARED((N, HID), jnp.float32),
            pltpu.VMEM_SHARED((N, HEADS), jnp.float32),
            pltpu.SemaphoreType.DMA,
            pltpu.SemaphoreType.DMA,
            pltpu.SemaphoreType.DMA,
        ],
    )
    return f(kv_tab, q_tab, rel_table, src3, dst3, feat3, zwv, zz)


def _post_body(
    wv_ref, z_ref, x_ref, r_ref, wo_ref, bo_ref, w1_ref, b1_ref, w2_ref,
    b2_ref, g1_ref, be1_ref, g2_ref, be2_ref, out_ref,
):
    acc = wv_ref[0] + wv_ref[1]
    zz = z_ref[0] + z_ref[1]
    zrep = jnp.dot(1.0 / zz, r_ref[...], preferred_element_type=jnp.float32)
    o = acc * zrep
    h1 = (
        x_ref[...]
        + jnp.dot(o, wo_ref[...], preferred_element_type=jnp.float32)
        + bo_ref[...]
    )
    m1 = jnp.mean(h1, axis=1, keepdims=True)
    v1 = jnp.mean((h1 - m1) ** 2, axis=1, keepdims=True)
    out1 = g1_ref[...] * (h1 - m1) / jnp.sqrt(v1 + 1e-5) + be1_ref[...]
    ff = jnp.maximum(
        jnp.dot(out1, w1_ref[...], preferred_element_type=jnp.float32)
        + b1_ref[...],
        0.0,
    )
    h2 = (
        out1
        + jnp.dot(ff, w2_ref[...], preferred_element_type=jnp.float32)
        + b2_ref[...]
    )
    m2 = jnp.mean(h2, axis=1, keepdims=True)
    v2 = jnp.mean((h2 - m2) ** 2, axis=1, keepdims=True)
    out_ref[...] = g2_ref[...] * (h2 - m2) / jnp.sqrt(v2 + 1e-5) + be2_ref[...]


def _post(wv2, z2, x, R, Wo, bo, W1, b1, W2, b2, g1, be1, g2, be2):
    full = lambda shape: pl.BlockSpec(shape, lambda i, _s=shape: tuple(0 for _ in _s))
    return pl.pallas_call(
        _post_body,
        grid=(_GRID,),
        in_specs=[
            pl.BlockSpec((2, _BLK, HID), lambda i: (0, i, 0)),
            pl.BlockSpec((2, _BLK, HEADS), lambda i: (0, i, 0)),
            pl.BlockSpec((_BLK, HID), lambda i: (i, 0)),
            full((HEADS, HID)),
            full((HID, HID)),
            full((1, HID)),
            full((HID, FF)),
            full((1, FF)),
            full((FF, HID)),
            full((1, HID)),
            full((1, HID)),
            full((1, HID)),
            full((1, HID)),
            full((1, HID)),
        ],
        out_specs=pl.BlockSpec((_BLK, HID), lambda i: (i, 0)),
        out_shape=jax.ShapeDtypeStruct((N, HID), jnp.float32),
    )(wv2, z2, x, R, Wo, bo, W1, b1, W2, b2, g1, be1, g2, be2)


def kernel(x, edge_index, edge_feat, rel_table, Wq, bq, Wk, Wv, Wo, bo,
           W1, b1, W2, b2, ln1_g, ln1_b, ln2_g, ln2_b):
    ei = edge_index.astype(jnp.int32)
    ef = edge_feat.astype(jnp.int32)
    src3 = ei[0].reshape(NWORK, NCHUNK, CHUNK)
    dst3 = ei[1].reshape(NWORK, NCHUNK, CHUNK)
    feat3 = ef.reshape(NWORK, NCHUNK, CHUNK)
    Wkv = jnp.concatenate([Wk, Wv], axis=1)

    q_tab, kv_tab = _qkv(x, Wq, bq.reshape(1, HID), Wkv)

    zwv = jnp.zeros((ROWS_PER_SUB, HID), jnp.float32)
    zz = jnp.zeros((ROWS_PER_SUB, HEADS), jnp.float32)
    lg4, wv2, z2 = _edge(kv_tab, q_tab, rel_table, src3, dst3, feat3, zwv, zz)
    lg_x = lg4.reshape(E, DK)

    R = jnp.repeat(jnp.eye(HEADS, dtype=jnp.float32), DK, axis=1)
    out_x = _post(
        wv2, z2, x, R, Wo, bo.reshape(1, HID), W1, b1.reshape(1, FF),
        W2, b2.reshape(1, HID), ln1_g.reshape(1, HID), ln1_b.reshape(1, HID),
        ln2_g.reshape(1, HID), ln2_b.reshape(1, HID),
    )
    return (out_x, lg_x)
